# baseline (speedup 1.0000x reference)
import jax
import jax.numpy as jnp
from jax.experimental import pallas as pl

NT, NP, NR = 131072, 18432, 18432
HID, OUT = 128, 2

_F32 = jnp.float32


def _combine(aggs, ws, x, wroot, bias, wzs, relu, blk):
    n = x.shape[0]
    na = len(aggs)
    nz = len(wzs)

    def body(*refs):
        arefs = refs[:na]
        wrefs = refs[na:2 * na]
        x_ref, wroot_ref, bias_ref = refs[2 * na:2 * na + 3]
        wzrefs = refs[2 * na + 3:2 * na + 3 + nz]
        h_ref = refs[2 * na + 3 + nz]
        zrefs = refs[2 * na + 4 + nz:]
        acc = jnp.dot(x_ref[...], wroot_ref[...],
                      preferred_element_type=_F32)
        for ar, wr in zip(arefs, wrefs):
            acc = acc + jnp.dot(ar[...], wr[...],
                                preferred_element_type=_F32)
        acc = acc + bias_ref[...]
        if relu:
            acc = jnp.maximum(acc, 0.0)
        h_ref[...] = acc
        for wzr, zr in zip(wzrefs, zrefs):
            zr[...] = jnp.dot(acc, wzr[...], preferred_element_type=_F32)

    hid = wroot.shape[1]
    in_specs = (
        [pl.BlockSpec((blk, a.shape[1]), lambda i: (i, 0)) for a in aggs]
        + [pl.BlockSpec(w.shape, lambda i: (0, 0)) for w in ws]
        + [pl.BlockSpec((blk, x.shape[1]), lambda i: (i, 0)),
           pl.BlockSpec(wroot.shape, lambda i: (0, 0)),
           pl.BlockSpec(bias.shape, lambda i: (0, 0))]
        + [pl.BlockSpec(wz.shape, lambda i: (0, 0)) for wz in wzs]
    )
    out_specs = ([pl.BlockSpec((blk, hid), lambda i: (i, 0))]
                 + [pl.BlockSpec((blk, 8), lambda i: (i, 0))] * nz)
    out_shape = ([jax.ShapeDtypeStruct((n, hid), _F32)]
                 + [jax.ShapeDtypeStruct((n, 8), _F32)] * nz)
    res = pl.pallas_call(
        body, grid=(n // blk,), in_specs=in_specs, out_specs=out_specs,
        out_shape=out_shape,
    )(*aggs, *ws, x, wroot, bias, *wzs)
    return res[0], res[1:]


def _final(terms, bias8, h, wroot8, blk, maxterm=None, wdep=None):
    n = h.shape[0]
    nt = len(terms)
    has_max = maxterm is not None

    def body(*refs):
        trefs = refs[:nt]
        bias_ref = refs[nt]
        idx = nt + 1
        if has_max:
            max_ref, wdep_ref = refs[idx], refs[idx + 1]
            idx += 2
        h_ref, wroot_ref, o_ref = refs[idx], refs[idx + 1], refs[idx + 2]
        acc = bias_ref[...] + jnp.zeros((blk, 8), _F32)
        for tr in trefs:
            acc = acc + tr[...]
        if has_max:
            acc = acc + jnp.dot(max_ref[...], wdep_ref[...],
                                preferred_element_type=_F32)
        acc = acc + jnp.dot(h_ref[...], wroot_ref[...],
                            preferred_element_type=_F32)
        o_ref[...] = acc[:, :OUT]

    in_specs = ([pl.BlockSpec((blk, 8), lambda i: (i, 0))] * nt
                + [pl.BlockSpec((1, 8), lambda i: (0, 0))])
    args = list(terms) + [bias8]
    if has_max:
        in_specs += [pl.BlockSpec((blk, HID), lambda i: (i, 0)),
                     pl.BlockSpec((HID, 8), lambda i: (0, 0))]
        args += [maxterm, wdep]
    in_specs += [pl.BlockSpec((blk, HID), lambda i: (i, 0)),
                 pl.BlockSpec((HID, 8), lambda i: (0, 0))]
    args += [h, wroot8]
    return pl.pallas_call(
        body, grid=(n // blk,), in_specs=in_specs,
        out_specs=pl.BlockSpec((blk, OUT), lambda i: (i, 0)),
        out_shape=jax.ShapeDtypeStruct((n, OUT), _F32),
    )(*args)


def kernel(x_task, x_pe, edges, params, router_emb):
    bs = x_pe.shape[0] // 9
    x_rtr = jnp.tile(router_emb, (bs, 1))
    x = {'task': x_task, 'pe': x_pe, 'router': x_rtr}
    ndst = {'task': NT, 'pe': NP, 'router': NR}

    def ssum(rel, src, d):
        ei = edges[rel]
        return jax.ops.segment_sum(x[src][ei[0]], ei[1],
                                   num_segments=ndst[d])

    ei = edges['depends_on']
    a = jax.ops.segment_max(x_task[ei[0]], ei[1], num_segments=NT)
    agg = {
        'depends_on': jnp.where(jnp.isneginf(a), 0.0, a),
        'rev_depends_on': ssum('rev_depends_on', 'task', 'task'),
        'rev_mapped_to': ssum('rev_mapped_to', 'pe', 'task'),
        'mapped_to': ssum('mapped_to', 'task', 'pe'),
        'rev_interface': ssum('rev_interface', 'pe', 'router'),
        'link': ssum('link', 'router', 'router'),
        'interface': ssum('interface', 'router', 'pe'),
    }

    lp = params[0]
    lp1 = params[1]

    def zw(rel):
        return jnp.pad(lp1[rel]['W_rel'], ((0, 0), (0, 8 - OUT)))

    def comb0(rels, x_d, zrels, blk):
        ws = [lp[r]['W_rel'] for r in rels]
        wroot = sum(lp[r]['W_root'] for r in rels)
        bias = sum(lp[r]['b_rel'] for r in rels).reshape(1, HID)
        return _combine([agg[r] for r in rels], ws, x_d, wroot, bias,
                        [zw(r) for r in zrels], True, blk)

    h_task, (z_rd, z_mp) = comb0(
        ['depends_on', 'rev_depends_on', 'rev_mapped_to'], x_task,
        ['rev_depends_on', 'mapped_to'], 1024)
    h_pe, (z_rm, z_ri) = comb0(
        ['mapped_to', 'interface'], x_pe,
        ['rev_mapped_to', 'rev_interface'], 512)
    h_rtr, (z_lk, z_if) = comb0(
        ['link', 'rev_interface'], x_rtr,
        ['link', 'interface'], 512)

    z = {'rev_depends_on': z_rd, 'mapped_to': z_mp, 'rev_mapped_to': z_rm,
         'rev_interface': z_ri, 'link': z_lk, 'interface': z_if}

    def zsum(rel, d):
        ei1 = edges[rel]
        return jax.ops.segment_sum(z[rel][ei1[0]], ei1[1],
                                   num_segments=ndst[d])

    o1_rd = zsum('rev_depends_on', 'task')
    o1_rm = zsum('rev_mapped_to', 'task')
    o1_mp = zsum('mapped_to', 'pe')
    o1_if = zsum('interface', 'pe')
    o1_lk = zsum('link', 'router')
    o1_ri = zsum('rev_interface', 'router')

    a1 = jax.ops.segment_max(h_task[ei[0]], ei[1], num_segments=NT)
    aggmax1 = jnp.where(jnp.isneginf(a1), 0.0, a1)

    def fin(add_terms, rels, h_d, blk, max_term=None):
        bias8 = sum(
            jnp.pad(lp1[r]['b_rel'], (0, 8 - OUT)) for r in rels
        ).reshape(1, 8)
        wroot8 = jnp.pad(sum(lp1[r]['W_root'] for r in rels),
                         ((0, 0), (0, 8 - OUT)))
        wdep = None
        if max_term is not None:
            pd = lp1['depends_on']
            bias8 = bias8 + jnp.pad(pd['b_rel'], (0, 8 - OUT)).reshape(1, 8)
            wroot8 = wroot8 + jnp.pad(pd['W_root'], ((0, 0), (0, 8 - OUT)))
            wdep = jnp.pad(pd['W_rel'], ((0, 0), (0, 8 - OUT)))
        return _final(add_terms, bias8, h_d, wroot8, blk,
                      maxterm=max_term, wdep=wdep)

    out_task = fin([o1_rd, o1_rm], ['rev_depends_on', 'rev_mapped_to'],
                   h_task, 1024, aggmax1)
    out_pe = fin([o1_mp, o1_if], ['mapped_to', 'interface'], h_pe, 512)
    out_rtr = fin([o1_lk, o1_ri], ['link', 'rev_interface'], h_rtr, 512)

    return (out_task, out_pe, out_rtr)
